# baseline (speedup 1.0000x reference)
import functools

import jax
import jax.numpy as jnp
from jax import lax
from jax.experimental import pallas as pl
from jax.experimental.pallas import tpu as pltpu
from jax.experimental.pallas import tpu_sc as plsc

_VOCAB = 1000000
_D = 32
_B = 16384

_NC = 2
_NS = 16
_NW = _NC * _NS
_BPW = _B // _NW
_GCHUNK = 128
_NG = _BPW // _GCHUNK
_L = 16


def _body(x_hbm, w_hbm, out_hbm, x_v, idx_v, rows_v, sem):
    wid = lax.axis_index("s") * _NC + lax.axis_index("c")
    base = wid * _BPW

    pltpu.sync_copy(x_hbm, x_v)

    def red(i, carry):
        mn, mx = carry
        v = x_v[pl.ds(i * _L, _L)]
        return jnp.minimum(mn, v), jnp.maximum(mx, v)

    v0 = x_v[pl.ds(0, _L)]
    mn_v, mx_v = lax.fori_loop(1, _B // _L, red, (v0, v0))

    min_val = mn_v[0]
    max_val = mx_v[0]
    for i in range(1, _L):
        min_val = jnp.minimum(min_val, mn_v[i])
        max_val = jnp.maximum(max_val, mx_v[i])

    denom = (max_val - min_val).astype(jnp.float32)
    max_int = jnp.float32(_B - 1)

    for j in range(_NG):
        for k in range(_GCHUNK // _L):
            off = j * _GCHUNK + k * _L
            xv = x_v[pl.ds(base + off, _L)]
            d = (xv - min_val).astype(jnp.float32)
            idx = ((d / denom) * max_int).astype(jnp.int32)
            idx_v[j, pl.ds(k * _L, _L)] = idx

    copies = [
        pltpu.async_copy(
            w_hbm.at[idx_v.at[j]],
            rows_v.at[pl.ds(j * _GCHUNK, _GCHUNK)],
            sem,
        )
        for j in range(_NG)
    ]
    for c in copies:
        c.wait()

    pltpu.sync_copy(rows_v, out_hbm.at[pl.ds(base, _BPW)])


@jax.jit
def _embed(x, w):
    mesh = plsc.VectorSubcoreMesh(core_axis_name="c", subcore_axis_name="s")
    f = functools.partial(
        pl.kernel,
        out_type=jax.ShapeDtypeStruct((_B, _D), jnp.float32),
        mesh=mesh,
        compiler_params=pltpu.CompilerParams(use_tc_tiling_on_sc=False),
        scratch_types=[
            pltpu.VMEM((_B,), jnp.int32),
            pltpu.VMEM((_NG, _GCHUNK), jnp.int32),
            pltpu.VMEM((_BPW, _D), jnp.float32),
            pltpu.SemaphoreType.DMA,
        ],
    )(_body)
    return f(x, w)


def kernel(x, W):
    return _embed(x, W)
